# baseline (speedup 1.0000x reference)
import functools

import jax
import jax.numpy as jnp
from jax import lax
from jax.experimental import pallas as pl
from jax.experimental.pallas import tpu as pltpu
from jax.experimental.pallas import tpu_sc as plsc

_SCALE = 10.0
_D = 128
_CHUNK = 128
_NCHUNK = 26
_B = 4096 * 26


def _make_kernel():
    info = plsc.get_sparse_core_info()
    nc, ns = info.num_cores, info.num_subcores
    nw = nc * ns
    assert _B == nw * _NCHUNK * _CHUNK

    mesh = plsc.VectorSubcoreMesh(core_axis_name="c", subcore_axis_name="s")

    @functools.partial(
        pl.kernel,
        mesh=mesh,
        out_type=jax.ShapeDtypeStruct((_B, _D), jnp.float32),
        scratch_types=[
            pltpu.VMEM((_NCHUNK, _CHUNK), jnp.int32),
            pltpu.VMEM((_CHUNK, _D), jnp.float32),
            pltpu.SemaphoreType.DMA,
        ],
    )
    def k(x_hbm, w_hbm, out_hbm, idx_v, buf, sem):
        wid = lax.axis_index("s") * nc + lax.axis_index("c")
        base = wid * (_NCHUNK * _CHUNK)
        pltpu.sync_copy(x_hbm.at[wid], idx_v)

        def chunk_body(c, carry):
            pltpu.async_copy(w_hbm.at[idx_v.at[c]], buf, sem).wait()

            def row_body(i, carry2):
                for j in range(_D // 16):
                    sl = pl.ds(j * 16, 16)
                    buf[i, sl] = buf[i, sl] * _SCALE
                return carry2

            lax.fori_loop(0, _CHUNK, row_body, 0)
            pltpu.sync_copy(buf, out_hbm.at[pl.ds(base + c * _CHUNK, _CHUNK)])
            return carry

        lax.fori_loop(0, _NCHUNK, chunk_body, 0)

    return k


_kernel_call = _make_kernel()


def kernel(x, weight):
    nw = _B // (_NCHUNK * _CHUNK)
    x_r = x.astype(jnp.int32).reshape(nw, _NCHUNK, _CHUNK)
    out = _kernel_call(x_r, weight)
    return out.reshape(x.shape[0], x.shape[1], _D)


# speedup vs baseline: 1.1907x; 1.1907x over previous
import functools

import jax
import jax.numpy as jnp
from jax import lax
from jax.experimental import pallas as pl
from jax.experimental.pallas import tpu as pltpu
from jax.experimental.pallas import tpu_sc as plsc

_SCALE = 10.0
_D = 128
_CHUNK = 128
_NCHUNK = 26
_B = 4096 * 26


def _make_kernel():
    info = plsc.get_sparse_core_info()
    nc, ns = info.num_cores, info.num_subcores
    nw = nc * ns
    assert _B == nw * _NCHUNK * _CHUNK

    mesh = plsc.VectorSubcoreMesh(core_axis_name="c", subcore_axis_name="s")

    @functools.partial(
        pl.kernel,
        mesh=mesh,
        out_type=jax.ShapeDtypeStruct((_B, _D), jnp.float32),
        scratch_types=[
            pltpu.VMEM((_NCHUNK, _CHUNK), jnp.int32),
            pltpu.VMEM((_CHUNK, _D), jnp.float32),
            pltpu.VMEM((_CHUNK, _D), jnp.float32),
            pltpu.VMEM((_CHUNK, _D), jnp.float32),
            pltpu.VMEM((_CHUNK, _D), jnp.float32),
            pltpu.SemaphoreType.DMA,
            pltpu.SemaphoreType.DMA,
            pltpu.SemaphoreType.DMA,
            pltpu.SemaphoreType.DMA,
        ],
    )
    def k(x_hbm, w_hbm, out_hbm, idx_v, gb0, gb1, ob0, ob1,
          gs0, gs1, ss0, ss1):
        wid = lax.axis_index("s") * nc + lax.axis_index("c")
        base = wid * (_NCHUNK * _CHUNK)
        pltpu.sync_copy(x_hbm.at[wid], idx_v)

        gbufs, obufs = (gb0, gb1), (ob0, ob1)
        gsems, ssems = (gs0, gs1), (ss0, ss1)

        for b in range(2):
            pltpu.async_copy(w_hbm.at[idx_v.at[b]], gbufs[b], gsems[b])

        def step(g, carry):
            for b in range(2):
                c = 2 * g + b
                gb, ob, gs, ss = gbufs[b], obufs[b], gsems[b], ssems[b]
                pltpu.make_async_copy(w_hbm.at[idx_v.at[c]], gb, gs).wait()

                def rows(i, carry2):
                    for r in range(2):
                        for j in range(_D // 16):
                            sl = pl.ds(j * 16, 16)
                            ob[2 * i + r, sl] = gb[2 * i + r, sl] * _SCALE
                    return carry2

                lax.fori_loop(0, _CHUNK // 2, rows, 0, unroll=False)

                out_slice = out_hbm.at[pl.ds(base + c * _CHUNK, _CHUNK)]

                @pl.when(c >= 2)
                def _():
                    pltpu.make_async_copy(ob, out_slice, ss).wait()

                pltpu.async_copy(ob, out_slice, ss)

                @pl.when(c + 2 < _NCHUNK)
                def _():
                    pltpu.async_copy(w_hbm.at[idx_v.at[c + 2]], gb, gs)
            return carry

        lax.fori_loop(0, _NCHUNK // 2, step, 0, unroll=False)

        for b in range(2):
            c = _NCHUNK - 2 + b
            out_slice = out_hbm.at[pl.ds(base + c * _CHUNK, _CHUNK)]
            pltpu.make_async_copy(obufs[b], out_slice, ssems[b]).wait()

    return k


_kernel_call = _make_kernel()


def kernel(x, weight):
    nw = _B // (_NCHUNK * _CHUNK)
    x_r = x.astype(jnp.int32).reshape(nw, _NCHUNK, _CHUNK)
    out = _kernel_call(x_r, weight)
    return out.reshape(x.shape[0], x.shape[1], _D)


# speedup vs baseline: 1.8452x; 1.5497x over previous
import functools

import jax
import jax.numpy as jnp
from jax import lax
from jax.experimental import pallas as pl
from jax.experimental.pallas import tpu as pltpu
from jax.experimental.pallas import tpu_sc as plsc

_SCALE = 10.0
_D = 128
_S = 26
_V = 4096
_VCHUNK = 4
_B = _V * _S


def _make_kernel():
    info = plsc.get_sparse_core_info()
    nc, ns = info.num_cores, info.num_subcores
    nw = nc * ns
    v_per_w = _V // nw
    nchunk = v_per_w // _VCHUNK
    rows = _VCHUNK * _S

    mesh = plsc.VectorSubcoreMesh(core_axis_name="c", subcore_axis_name="s")

    @functools.partial(
        pl.kernel,
        mesh=mesh,
        out_type=jax.ShapeDtypeStruct((_V, _S, _D), jnp.float32),
        scratch_types=[
            pltpu.VMEM((v_per_w * _S,), jnp.int32),
            pltpu.VMEM((rows, _D), jnp.float32),
            pltpu.VMEM((rows, _D), jnp.float32),
            pltpu.VMEM((_VCHUNK, _S, _D), jnp.float32),
            pltpu.VMEM((_VCHUNK, _S, _D), jnp.float32),
            pltpu.SemaphoreType.DMA,
            pltpu.SemaphoreType.DMA,
            pltpu.SemaphoreType.DMA,
            pltpu.SemaphoreType.DMA,
        ],
        compiler_params=pltpu.CompilerParams(use_tc_tiling_on_sc=True),
    )
    def k(x_hbm, w_hbm, out_hbm, idx_v, gb0, gb1, ob0, ob1,
          gs0, gs1, ss0, ss1):
        wid = lax.axis_index("s") * nc + lax.axis_index("c")
        vbase = wid * v_per_w
        pltpu.sync_copy(x_hbm.at[pl.ds(vbase * _S, v_per_w * _S)], idx_v)

        gbufs, obufs = (gb0, gb1), (ob0, ob1)
        gsems, ssems = (gs0, gs1), (ss0, ss1)

        def gather(c, b):
            pltpu.async_copy(
                w_hbm.at[idx_v.at[pl.ds(c * rows, rows)]], gbufs[b], gsems[b])

        for b in range(2):
            gather(b, b)

        def step(g, carry):
            for b in range(2):
                c = 2 * g + b
                gb, ob, gs, ss = gbufs[b], obufs[b], gsems[b], ssems[b]
                pltpu.make_async_copy(
                    w_hbm.at[idx_v.at[pl.ds(c * rows, rows)]], gb, gs).wait()

                for vv in range(_VCHUNK):
                    def srow(s, carry2, vv=vv):
                        for j in range(_D // 16):
                            sl = pl.ds(j * 16, 16)
                            ob[vv, s, sl] = gb[vv * _S + s, sl] * _SCALE
                        return carry2

                    lax.fori_loop(0, _S, srow, 0, unroll=False)

                out_slice = out_hbm.at[pl.ds(vbase + c * _VCHUNK, _VCHUNK)]

                @pl.when(c >= 2)
                def _():
                    pltpu.make_async_copy(ob, out_slice, ss).wait()

                pltpu.async_copy(ob, out_slice, ss)

                @pl.when(c + 2 < nchunk)
                def _():
                    gather(c + 2, b)
            return carry

        lax.fori_loop(0, nchunk // 2, step, 0, unroll=False)

        for b in range(2):
            c = nchunk - 2 + b
            out_slice = out_hbm.at[pl.ds(vbase + c * _VCHUNK, _VCHUNK)]
            pltpu.make_async_copy(obufs[b], out_slice, ssems[b]).wait()

    return k


_kernel_call = _make_kernel()


def kernel(x, weight):
    x_flat = x.astype(jnp.int32).reshape(_B)
    return _kernel_call(x_flat, weight)


# speedup vs baseline: 3.3937x; 1.8393x over previous
import functools

import jax
import jax.numpy as jnp
from jax import lax
from jax.experimental import pallas as pl
from jax.experimental.pallas import tpu as pltpu
from jax.experimental.pallas import tpu_sc as plsc

_SCALE = 10.0
_D = 128
_S = 26
_V = 4096
_B = _V * _S


def _make_kernel():
    info = plsc.get_sparse_core_info()
    nc, ns = info.num_cores, info.num_subcores
    nw = nc * ns
    v_per_w = _V // nw

    mesh = plsc.VectorSubcoreMesh(core_axis_name="c", subcore_axis_name="s")

    @functools.partial(
        pl.kernel,
        mesh=mesh,
        out_type=jax.ShapeDtypeStruct((_B, _D), jnp.float32),
        scratch_types=[
            pltpu.VMEM((_S, v_per_w), jnp.int32),
            pltpu.VMEM((v_per_w, _D), jnp.float32),
            pltpu.VMEM((v_per_w, _D), jnp.float32),
            pltpu.VMEM((v_per_w, _D), jnp.float32),
            pltpu.VMEM((v_per_w, _D), jnp.float32),
            pltpu.SemaphoreType.DMA,
            pltpu.SemaphoreType.DMA,
            pltpu.SemaphoreType.DMA,
            pltpu.SemaphoreType.DMA,
        ],
    )
    def k(xt_hbm, w_hbm, out_hbm, idx_v, gb0, gb1, ob0, ob1,
          gs0, gs1, ss0, ss1):
        wid = lax.axis_index("s") * nc + lax.axis_index("c")
        vbase = wid * v_per_w
        pltpu.sync_copy(xt_hbm.at[:, pl.ds(vbase, v_per_w)], idx_v)

        gbufs, obufs = (gb0, gb1), (ob0, ob1)
        gsems, ssems = (gs0, gs1), (ss0, ss1)

        def gather(c, b):
            pltpu.async_copy(w_hbm.at[idx_v.at[c]], gbufs[b], gsems[b])

        for b in range(2):
            gather(b, b)

        def step(g, carry):
            for b in range(2):
                c = 2 * g + b
                gb, ob, gs, ss = gbufs[b], obufs[b], gsems[b], ssems[b]
                pltpu.make_async_copy(w_hbm.at[idx_v.at[c]], gb, gs).wait()

                def rows(i, carry2):
                    for r in range(2):
                        for j in range(_D // 16):
                            sl = pl.ds(j * 16, 16)
                            ob[2 * i + r, sl] = gb[2 * i + r, sl] * _SCALE
                    return carry2

                lax.fori_loop(0, v_per_w // 2, rows, 0, unroll=False)

                out_slice = out_hbm.at[pl.ds(c * _V + vbase, v_per_w)]

                @pl.when(c >= 2)
                def _():
                    pltpu.make_async_copy(ob, out_slice, ss).wait()

                pltpu.async_copy(ob, out_slice, ss)

                @pl.when(c + 2 < _S)
                def _():
                    gather(c + 2, b)
            return carry

        lax.fori_loop(0, _S // 2, step, 0, unroll=False)

        for b in range(2):
            c = _S - 2 + b
            out_slice = out_hbm.at[pl.ds(c * _V + vbase, v_per_w)]
            pltpu.make_async_copy(obufs[b], out_slice, ssems[b]).wait()

    return k


_kernel_call = _make_kernel()


def kernel(x, weight):
    x_t = x.astype(jnp.int32).T
    out = _kernel_call(x_t, weight)
    return out.reshape(_S, _V, _D).transpose(1, 0, 2)
